# baseline (speedup 1.0000x reference)
import functools

import jax
import jax.numpy as jnp
from jax import lax
from jax.experimental import pallas as pl
from jax.experimental.pallas import tpu as pltpu
from jax.experimental.pallas import tpu_sc as plsc

_NC = 2
_NS = 16
_NW = _NC * _NS
_CB = 128


def _cdiv(a, b):
  return (a + b - 1) // b


def _mm2(x, wl, wr, b, rb):
  m, k = x.shape
  dl, dr = wl.shape[1], wr.shape[1]

  def body(x_ref, wl_ref, wr_ref, b_ref, ol_ref, or_ref):
    xv = x_ref[...]
    ol_ref[...] = jnp.dot(xv, wl_ref[...], preferred_element_type=jnp.float32)
    or_ref[...] = jnp.dot(xv, wr_ref[...],
                          preferred_element_type=jnp.float32) + b_ref[...]

  return pl.pallas_call(
      body,
      grid=(m // rb,),
      in_specs=[
          pl.BlockSpec((rb, k), lambda i: (i, 0)),
          pl.BlockSpec((k, dl), lambda i: (0, 0)),
          pl.BlockSpec((k, dr), lambda i: (0, 0)),
          pl.BlockSpec((1, dr), lambda i: (0, 0)),
      ],
      out_specs=[
          pl.BlockSpec((rb, dl), lambda i: (i, 0)),
          pl.BlockSpec((rb, dr), lambda i: (i, 0)),
      ],
      out_shape=[
          jax.ShapeDtypeStruct((m, dl), jnp.float32),
          jax.ShapeDtypeStruct((m, dr), jnp.float32),
      ],
  )(x, wl, wr, b)


def _combine_proj(a0, a1, d0, d1, r1b, w2l, w2r, w3, b2l, b3, rb):
  m, dh = r1b.shape

  def body(a0_ref, a1_ref, d0_ref, d1_ref, r_ref, w2l_ref, w2r_ref, w3_ref,
           b2l_ref, b3_ref, pu_ref, pv_ref, rcu_ref, rcv_ref, rd_ref):
    rec = 1.0 / jnp.maximum(d0_ref[...] + d1_ref[...], 1.0)
    h = jnp.maximum((a0_ref[...] + a1_ref[...]) * rec[:, :1] + r_ref[...],
                    0.0)
    w3a = w3_ref[0:16, :]
    w3b = w3_ref[16:32, :]
    dot = lambda p, q: jnp.dot(p, q, preferred_element_type=jnp.float32)
    pu_ref[...] = dot(h, dot(w2l_ref[...], w3a))
    pv_ref[...] = dot(h, dot(w2l_ref[...], w3b))
    rcu_ref[...] = (dot(h, dot(w2r_ref[...], w3a))
                    + dot(b2l_ref[...], w3a) + b3_ref[...])
    rcv_ref[...] = dot(h, dot(w2r_ref[...], w3b)) + dot(b2l_ref[...], w3b)
    rd_ref[...] = rec

  out16 = jax.ShapeDtypeStruct((m, 16), jnp.float32)
  return pl.pallas_call(
      body,
      grid=(m // rb,),
      in_specs=[
          pl.BlockSpec((rb, dh), lambda i: (i, 0)),
          pl.BlockSpec((rb, dh), lambda i: (i, 0)),
          pl.BlockSpec((rb, 16), lambda i: (i, 0)),
          pl.BlockSpec((rb, 16), lambda i: (i, 0)),
          pl.BlockSpec((rb, dh), lambda i: (i, 0)),
          pl.BlockSpec((dh, 16), lambda i: (0, 0)),
          pl.BlockSpec((dh, 16), lambda i: (0, 0)),
          pl.BlockSpec((32, 16), lambda i: (0, 0)),
          pl.BlockSpec((1, 16), lambda i: (0, 0)),
          pl.BlockSpec((1, 16), lambda i: (0, 0)),
      ],
      out_specs=[pl.BlockSpec((rb, 16), lambda i: (i, 0))] * 5,
      out_shape=[out16] * 5,
  )(a0, a1, d0, d1, r1b, w2l, w2r, w3, b2l, b3)


def _decode(ccat, w4, b4, rb):
  m = ccat.shape[0] // 2
  nblk = m // rb

  def body(cu_ref, cv_ref, w4_ref, b4_ref, o_ref):
    hid = jnp.maximum(cu_ref[...] + cv_ref[...], 0.0)
    o_ref[...] = jnp.sum(hid * w4_ref[...], axis=1) + b4_ref[0, 0]

  return pl.pallas_call(
      body,
      grid=(nblk,),
      in_specs=[
          pl.BlockSpec((rb, 16), lambda i: (i, 0)),
          pl.BlockSpec((rb, 16), lambda i: (i + nblk, 0)),
          pl.BlockSpec((1, 16), lambda i: (0, 0)),
          pl.BlockSpec((1, 1), lambda i: (0, 0)),
      ],
      out_specs=pl.BlockSpec((rb,), lambda i: (i,)),
      out_shape=jax.ShapeDtypeStruct((m,), jnp.float32),
  )(ccat, ccat, w4, b4)


def _zero_rows(buf, rows, d):
  z = jnp.zeros((16,), jnp.float32)

  def zf(i, c):
    for kk in range(d // 16):
      buf[i, pl.ds(kk * 16, 16)] = z
    return c

  lax.fori_loop(0, rows, zf, 0)


def _seg_sum(p, src3, dst3, n_pad, with_deg):
  d = p.shape[1]
  nw, nchunk, cb = src3.shape
  assert nchunk % 2 == 0 and nchunk >= 4
  rpt = n_pad // _NS
  mesh = plsc.VectorSubcoreMesh(core_axis_name="c", subcore_axis_name="s",
                                num_cores=_NC, num_subcores=_NS)

  out_type = [jax.ShapeDtypeStruct((_NC * n_pad, d), jnp.float32)]
  scratch = [
      pltpu.VMEM_SHARED((n_pad, d), jnp.float32),
      pltpu.VMEM((cb, d), jnp.float32),
      pltpu.VMEM((cb, d), jnp.float32),
      pltpu.VMEM((nchunk, cb), jnp.int32),
      pltpu.VMEM((nchunk, cb), jnp.int32),
      pltpu.SemaphoreType.DMA,
      pltpu.SemaphoreType.DMA,
  ]
  if with_deg:
    out_type.append(jax.ShapeDtypeStruct((_NC * n_pad, 16), jnp.float32))
    scratch += [
        pltpu.VMEM_SHARED((n_pad, 16), jnp.float32),
        pltpu.VMEM((cb, 16), jnp.float32),
        pltpu.SemaphoreType.DMA,
    ]

  @functools.partial(
      pl.kernel,
      out_type=out_type,
      mesh=mesh,
      compiler_params=pltpu.CompilerParams(use_tc_tiling_on_sc=False),
      scratch_types=scratch,
  )
  def k(p_hbm, src_hbm, dst_hbm, agg_out, *rest):
    if with_deg:
      (deg_out, agg_sp, g0, g1, sidx, didx, s0, s1, deg_sp, ones, dsem) = rest
    else:
      (agg_sp, g0, g1, sidx, didx, s0, s1) = rest
    c = lax.axis_index("c")
    s = lax.axis_index("s")
    w = s * _NC + c

    _zero_rows(g0, cb, d)
    for kk in range(rpt // cb):
      pltpu.sync_copy(g0, agg_sp.at[pl.ds(s * rpt + kk * cb, cb)])
    if with_deg:
      _zero_rows(ones, cb, 16)
      for kk in range(rpt // cb):
        pltpu.sync_copy(ones, deg_sp.at[pl.ds(s * rpt + kk * cb, cb)])
      one = jnp.ones((16,), jnp.float32)

      def of(i, cacc):
        ones[i, pl.ds(0, 16)] = one
        return cacc

      lax.fori_loop(0, cb, of, 0)

    pltpu.sync_copy(src_hbm.at[w], sidx)
    pltpu.sync_copy(dst_hbm.at[w], didx)
    plsc.subcore_barrier()

    bufs = (g0, g1)
    sems = (s0, s1)

    def start(j, b):
      pltpu.async_copy(p_hbm.at[sidx.at[j]], bufs[b], sems[b])

    def finish(j, b):
      pltpu.make_async_copy(p_hbm.at[sidx.at[j]], bufs[b], sems[b]).wait()
      pltpu.sync_copy(bufs[b], agg_sp.at[didx.at[j]], add=True)
      if with_deg:
        pltpu.async_copy(ones, deg_sp.at[didx.at[j]], dsem, add=True)

    start(0, 0)

    @pl.loop(0, nchunk - 2, step=2)
    def body(j):
      start(j + 1, 1)
      finish(j, 0)
      start(j + 2, 0)
      finish(j + 1, 1)

    start(nchunk - 1, 1)
    finish(nchunk - 2, 0)
    finish(nchunk - 1, 1)

    if with_deg:
      def drain(j, cacc):
        pltpu.make_async_copy(ones, deg_sp.at[didx.at[0]], dsem).wait()
        return cacc

      lax.fori_loop(0, nchunk, drain, 0)

    plsc.subcore_barrier()

    pltpu.sync_copy(agg_sp.at[pl.ds(s * rpt, rpt)],
                    agg_out.at[pl.ds(c * n_pad + s * rpt, rpt)])
    if with_deg:
      pltpu.sync_copy(deg_sp.at[pl.ds(s * rpt, rpt)],
                      deg_out.at[pl.ds(c * n_pad + s * rpt, rpt)])

  return k(p, src3, dst3)


def _fused_uv(pcs, rc2, rd, srcA, dst2, eidx2, n, n_pad):
  _, nchunk, cb = dst2.shape
  assert nchunk % 2 == 0 and nchunk >= 4
  epw = nchunk * cb
  e_pad = _NS * epw
  rpt = n_pad // _NS
  nblk = rpt // cb
  mesh = plsc.VectorSubcoreMesh(core_axis_name="c", subcore_axis_name="s",
                                num_cores=_NC, num_subcores=_NS)

  @functools.partial(
      pl.kernel,
      out_type=[jax.ShapeDtypeStruct((2 * e_pad, 16), jnp.float32)],
      mesh=mesh,
      compiler_params=pltpu.CompilerParams(use_tc_tiling_on_sc=False),
      scratch_types=[
          pltpu.VMEM_SHARED((n_pad, 16), jnp.float32),
          pltpu.VMEM((nchunk, cb), jnp.int32),
          pltpu.VMEM((nchunk, cb), jnp.int32),
          pltpu.VMEM((nchunk, cb), jnp.int32),
          pltpu.VMEM((cb, 16), jnp.float32),
          pltpu.VMEM((cb, 16), jnp.float32),
          pltpu.VMEM((cb, 16), jnp.float32),
          pltpu.VMEM((cb, 16), jnp.float32),
          pltpu.VMEM((cb, 16), jnp.float32),
          pltpu.VMEM((cb, 16), jnp.float32),
          pltpu.SemaphoreType.DMA,
          pltpu.SemaphoreType.DMA,
          pltpu.SemaphoreType.DMA,
          pltpu.SemaphoreType.DMA,
      ],
  )
  def k(pcs_hbm, rc2_hbm, rd_hbm, srcA_hbm, dst2_hbm, eidx_hbm, cc_out,
        tab, sidx, didx, eidx, g0, g1, u0, u1, rcb, rdb,
        s0, s1, w0, w1):
    c = lax.axis_index("c")
    s = lax.axis_index("s")

    _zero_rows(g0, cb, 16)
    for kk in range(nblk):
      pltpu.sync_copy(g0, tab.at[pl.ds(s * rpt + kk * cb, cb)])

    pltpu.sync_copy(srcA_hbm.at[c * _NS + s], sidx)
    pltpu.sync_copy(dst2_hbm.at[s], didx)
    pltpu.sync_copy(eidx_hbm.at[c * _NS + s], eidx)
    plsc.subcore_barrier()

    bufs = (g0, g1)
    sems = (s0, s1)

    def start(j, b):
      pltpu.async_copy(pcs_hbm.at[sidx.at[j]], bufs[b], sems[b])

    def finish(j, b):
      pltpu.make_async_copy(pcs_hbm.at[sidx.at[j]], bufs[b], sems[b]).wait()
      pltpu.sync_copy(bufs[b], tab.at[didx.at[j]], add=True)

    start(0, 0)

    @pl.loop(0, nchunk - 2, step=2)
    def seg_body(j):
      start(j + 1, 1)
      finish(j, 0)
      start(j + 2, 0)
      finish(j + 1, 1)

    start(nchunk - 1, 1)
    finish(nchunk - 2, 0)
    finish(nchunk - 1, 1)
    plsc.subcore_barrier()

    for blk in range(nblk):
      row0 = s * rpt + blk * cb
      pltpu.sync_copy(tab.at[pl.ds(row0, cb)], g0)
      pltpu.sync_copy(rc2_hbm.at[pl.ds(c * n_pad + row0, cb)], rcb)
      pltpu.sync_copy(rd_hbm.at[pl.ds(row0, cb)], rdb)

      def rowf(i, cacc):
        u0[i, pl.ds(0, 16)] = (g0[i, pl.ds(0, 16)] * rdb[i, pl.ds(0, 16)]
                               + rcb[i, pl.ds(0, 16)])
        return cacc

      lax.fori_loop(0, cb, rowf, 0)
      pltpu.sync_copy(u0, tab.at[pl.ds(row0, cb)])
    plsc.subcore_barrier()

    ubufs = (u0, u1)
    gsems = (s0, s1)
    wsems = (w0, w1)

    def start_gather(j, p):
      pltpu.async_copy(tab.at[eidx.at[j]], ubufs[p], gsems[p])

    def wait_gather(j, p):
      pltpu.make_async_copy(tab.at[eidx.at[j]], ubufs[p], gsems[p]).wait()

    def obase(j):
      return c * e_pad + s * epw + j * cb

    def start_write(j, p):
      pltpu.async_copy(ubufs[p], cc_out.at[pl.ds(obase(j), cb)], wsems[p])

    def wait_write(j, p):
      pltpu.make_async_copy(ubufs[p], cc_out.at[pl.ds(obase(j), cb)],
                            wsems[p]).wait()

    start_gather(0, 0)
    start_gather(1, 1)
    wait_gather(0, 0)
    start_write(0, 0)

    @pl.loop(1, nchunk - 1, step=2)
    def body(j):
      wait_write(j - 1, 0)
      start_gather(j + 1, 0)
      wait_gather(j, 1)
      start_write(j, 1)
      wait_write(j, 1)
      start_gather(j + 2, 1)
      wait_gather(j + 1, 0)
      start_write(j + 1, 0)

    wait_write(nchunk - 2, 0)
    wait_gather(nchunk - 1, 1)
    start_write(nchunk - 1, 1)
    wait_write(nchunk - 1, 1)

  return k(pcs, rc2, rd, srcA, dst2, eidx2)[0]


def kernel(x, edge_index, W1l, b1l, W1r, W2l, b2l, W2r, W3, b3, W4, b4):
  n, d_in = x.shape
  e = edge_index.shape[1]
  d_hid = W1l.shape[1]

  e_pad = _cdiv(e, 16384) * 16384
  epw = e_pad // _NW
  n_pad = _cdiv(n + 1, _NS * _CB) * _NS * _CB

  src = edge_index[0]
  dst = edge_index[1]
  pad = e_pad - e
  src_p = jnp.concatenate([src, jnp.zeros((pad,), jnp.int32)])
  dst_p = jnp.concatenate([dst, jnp.full((pad,), n, jnp.int32)])
  src3 = src_p.reshape(_NW, epw // _CB, _CB)
  dst3 = dst_p.reshape(_NW, epw // _CB, _CB)

  eps = e_pad // _NS
  src2 = src_p.reshape(_NS, eps // _CB, _CB)
  dst2 = dst_p.reshape(_NS, eps // _CB, _CB)
  srcA = jnp.concatenate([src2, src2 + n], axis=0)
  eidx2 = jnp.concatenate([src2, dst2], axis=0)

  p1, r1b = _mm2(x, W1l, W1r, b1l.reshape(1, d_hid), rb=2000)
  agg1, deg = _seg_sum(p1, src3, dst3, n_pad, with_deg=True)
  a10, a11 = agg1[:n], agg1[n_pad:n_pad + n]
  d0, d1 = deg[:n], deg[n_pad:n_pad + n]

  pu, pv, rcu, rcv, rd = _combine_proj(
      a10, a11, d0, d1, r1b, W2l, W2r, W3,
      b2l.reshape(1, 16), b3.reshape(1, 16), rb=2000)

  zp = jnp.zeros((n_pad - n, 16), jnp.float32)
  pcs = jnp.concatenate([pu, pv], axis=0)
  rc2 = jnp.concatenate([rcu, zp, rcv, zp], axis=0)
  rd_p = jnp.concatenate([rd, zp], axis=0)

  ccat = _fused_uv(pcs, rc2, rd_p, srcA, dst2, eidx2, n, n_pad)
  out = _decode(ccat, W4.reshape(1, 16), b4.reshape(1, 1), rb=16384)
  return out[:e]


# speedup vs baseline: 1.7418x; 1.7418x over previous
import functools

import jax
import jax.numpy as jnp
from jax import lax
from jax.experimental import pallas as pl
from jax.experimental.pallas import tpu as pltpu
from jax.experimental.pallas import tpu_sc as plsc

_NC = 2
_NS = 16
_NW = _NC * _NS
_CB = 128


def _cdiv(a, b):
  return (a + b - 1) // b


def _mm2(x, wl, wr, b, rb):
  m, k = x.shape
  dl, dr = wl.shape[1], wr.shape[1]

  def body(x_ref, wl_ref, wr_ref, b_ref, ol_ref, or_ref):
    xv = x_ref[...]
    ol_ref[...] = jnp.dot(xv, wl_ref[...], preferred_element_type=jnp.float32)
    or_ref[...] = jnp.dot(xv, wr_ref[...],
                          preferred_element_type=jnp.float32) + b_ref[...]

  return pl.pallas_call(
      body,
      grid=(m // rb,),
      in_specs=[
          pl.BlockSpec((rb, k), lambda i: (i, 0)),
          pl.BlockSpec((k, dl), lambda i: (0, 0)),
          pl.BlockSpec((k, dr), lambda i: (0, 0)),
          pl.BlockSpec((1, dr), lambda i: (0, 0)),
      ],
      out_specs=[
          pl.BlockSpec((rb, dl), lambda i: (i, 0)),
          pl.BlockSpec((rb, dr), lambda i: (i, 0)),
      ],
      out_shape=[
          jax.ShapeDtypeStruct((m, dl), jnp.float32),
          jax.ShapeDtypeStruct((m, dr), jnp.float32),
      ],
  )(x, wl, wr, b)


def _combine_proj(agg1, deg, r1b, w2l, w2r, w3, b2l, b3, n_pad, rb):
  dh = r1b.shape[1]
  nb = n_pad // rb
  noff = n_pad // rb

  def body(a0_ref, a1_ref, d0_ref, d1_ref, r_ref, w2l_ref, w2r_ref, w3_ref,
           b2l_ref, b3_ref, pcs_ref, rc_ref, rd_ref):
    rec = 1.0 / jnp.maximum(d0_ref[...] + d1_ref[...], 1.0)
    h = jnp.maximum((a0_ref[...] + a1_ref[...]) * rec[:, :1] + r_ref[...],
                    0.0)
    w3j = w3_ref[...]
    dot = lambda p, q: jnp.dot(p, q, preferred_element_type=jnp.float32)
    ucoef = 1.0 - pl.program_id(0).astype(jnp.float32)
    pcs_ref[0] = dot(h, dot(w2l_ref[...], w3j))
    rc_ref[0] = dot(h, dot(w2r_ref[...], w3j)) + dot(b2l_ref[...],
                                                     w3j) + b3_ref[...] * ucoef
    rd_ref[...] = rec

  out3 = jax.ShapeDtypeStruct((2, n_pad, 16), jnp.float32)
  return pl.pallas_call(
      body,
      grid=(2, nb),
      in_specs=[
          pl.BlockSpec((rb, dh), lambda j, i: (i, 0)),
          pl.BlockSpec((rb, dh), lambda j, i: (i + noff, 0)),
          pl.BlockSpec((rb, 16), lambda j, i: (i, 0)),
          pl.BlockSpec((rb, 16), lambda j, i: (i + noff, 0)),
          pl.BlockSpec((rb, dh), lambda j, i: (i, 0)),
          pl.BlockSpec((dh, 16), lambda j, i: (0, 0)),
          pl.BlockSpec((dh, 16), lambda j, i: (0, 0)),
          pl.BlockSpec((16, 16), lambda j, i: (j, 0)),
          pl.BlockSpec((1, 16), lambda j, i: (0, 0)),
          pl.BlockSpec((1, 16), lambda j, i: (0, 0)),
      ],
      out_specs=[
          pl.BlockSpec((1, rb, 16), lambda j, i: (j, i, 0)),
          pl.BlockSpec((1, rb, 16), lambda j, i: (j, i, 0)),
          pl.BlockSpec((rb, 16), lambda j, i: (i, 0)),
      ],
      out_shape=[out3, out3, jax.ShapeDtypeStruct((n_pad, 16), jnp.float32)],
  )(agg1, agg1, deg, deg, r1b, w2l, w2r, w3, b2l, b3)


def _decode(ccat2, w4t, sel, b4, rb8):
  m8 = ccat2.shape[0] // 2
  nblk = m8 // rb8

  def body(cu_ref, cv_ref, w4_ref, sel_ref, b4_ref, o_ref):
    hid = jnp.maximum(cu_ref[...] + cv_ref[...], 0.0)
    o_ref[...] = jnp.dot(hid * w4_ref[...], sel_ref[...],
                         preferred_element_type=jnp.float32,
                         precision=jax.lax.Precision.HIGHEST) + b4_ref[0, 0]

  return pl.pallas_call(
      body,
      grid=(nblk,),
      in_specs=[
          pl.BlockSpec((rb8, 128), lambda i: (i, 0)),
          pl.BlockSpec((rb8, 128), lambda i: (i + nblk, 0)),
          pl.BlockSpec((1, 128), lambda i: (0, 0)),
          pl.BlockSpec((128, 8), lambda i: (0, 0)),
          pl.BlockSpec((1, 1), lambda i: (0, 0)),
      ],
      out_specs=pl.BlockSpec((rb8, 8), lambda i: (i, 0)),
      out_shape=jax.ShapeDtypeStruct((m8, 8), jnp.float32),
  )(ccat2, ccat2, w4t, sel, b4)


def _zero_rows(buf, rows, d):
  z = jnp.zeros((16,), jnp.float32)

  def zf(i, c):
    for kk in range(d // 16):
      buf[i, pl.ds(kk * 16, 16)] = z
    return c

  lax.fori_loop(0, rows, zf, 0)


def _seg_sum(p, src3, dst3, n_pad, with_deg):
  d = p.shape[1]
  nw, nchunk, cb = src3.shape
  assert nchunk % 2 == 0 and nchunk >= 4
  rpt = n_pad // _NS
  mesh = plsc.VectorSubcoreMesh(core_axis_name="c", subcore_axis_name="s",
                                num_cores=_NC, num_subcores=_NS)

  out_type = [jax.ShapeDtypeStruct((_NC * n_pad, d), jnp.float32)]
  scratch = [
      pltpu.VMEM_SHARED((n_pad, d), jnp.float32),
      pltpu.VMEM((cb, d), jnp.float32),
      pltpu.VMEM((cb, d), jnp.float32),
      pltpu.VMEM((nchunk, cb), jnp.int32),
      pltpu.VMEM((nchunk, cb), jnp.int32),
      pltpu.SemaphoreType.DMA,
      pltpu.SemaphoreType.DMA,
  ]
  if with_deg:
    out_type.append(jax.ShapeDtypeStruct((_NC * n_pad, 16), jnp.float32))
    scratch += [
        pltpu.VMEM_SHARED((n_pad, 16), jnp.float32),
        pltpu.VMEM((cb, 16), jnp.float32),
        pltpu.SemaphoreType.DMA,
    ]

  @functools.partial(
      pl.kernel,
      out_type=out_type,
      mesh=mesh,
      compiler_params=pltpu.CompilerParams(use_tc_tiling_on_sc=False),
      scratch_types=scratch,
  )
  def k(p_hbm, src_hbm, dst_hbm, agg_out, *rest):
    if with_deg:
      (deg_out, agg_sp, g0, g1, sidx, didx, s0, s1, deg_sp, ones, dsem) = rest
    else:
      (agg_sp, g0, g1, sidx, didx, s0, s1) = rest
    c = lax.axis_index("c")
    s = lax.axis_index("s")
    w = s * _NC + c

    _zero_rows(g0, cb, d)
    for kk in range(rpt // cb):
      pltpu.sync_copy(g0, agg_sp.at[pl.ds(s * rpt + kk * cb, cb)])
    if with_deg:
      _zero_rows(ones, cb, 16)
      for kk in range(rpt // cb):
        pltpu.sync_copy(ones, deg_sp.at[pl.ds(s * rpt + kk * cb, cb)])
      one = jnp.ones((16,), jnp.float32)

      def of(i, cacc):
        ones[i, pl.ds(0, 16)] = one
        return cacc

      lax.fori_loop(0, cb, of, 0)

    pltpu.sync_copy(src_hbm.at[w], sidx)
    pltpu.sync_copy(dst_hbm.at[w], didx)
    plsc.subcore_barrier()

    bufs = (g0, g1)
    sems = (s0, s1)

    def start(j, b):
      pltpu.async_copy(p_hbm.at[sidx.at[j]], bufs[b], sems[b])

    def finish(j, b):
      pltpu.make_async_copy(p_hbm.at[sidx.at[j]], bufs[b], sems[b]).wait()
      pltpu.sync_copy(bufs[b], agg_sp.at[didx.at[j]], add=True)
      if with_deg:
        pltpu.async_copy(ones, deg_sp.at[didx.at[j]], dsem, add=True)

    start(0, 0)

    @pl.loop(0, nchunk - 2, step=2)
    def body(j):
      start(j + 1, 1)
      finish(j, 0)
      start(j + 2, 0)
      finish(j + 1, 1)

    start(nchunk - 1, 1)
    finish(nchunk - 2, 0)
    finish(nchunk - 1, 1)

    if with_deg:
      def drain(j, cacc):
        pltpu.make_async_copy(ones, deg_sp.at[didx.at[0]], dsem).wait()
        return cacc

      lax.fori_loop(0, nchunk, drain, 0)

    plsc.subcore_barrier()

    pltpu.sync_copy(agg_sp.at[pl.ds(s * rpt, rpt)],
                    agg_out.at[pl.ds(c * n_pad + s * rpt, rpt)])
    if with_deg:
      pltpu.sync_copy(deg_sp.at[pl.ds(s * rpt, rpt)],
                      deg_out.at[pl.ds(c * n_pad + s * rpt, rpt)])

  return k(p, src3, dst3)


def _fused_uv(pcs, rc2, rd, srcA, dst2, eidx2, n, n_pad):
  _, nchunk, cb = dst2.shape
  assert nchunk % 2 == 0 and nchunk >= 4
  epw = nchunk * cb
  e_pad = _NS * epw
  rpt = n_pad // _NS
  nblk = rpt // cb
  mesh = plsc.VectorSubcoreMesh(core_axis_name="c", subcore_axis_name="s",
                                num_cores=_NC, num_subcores=_NS)

  @functools.partial(
      pl.kernel,
      out_type=[jax.ShapeDtypeStruct((2 * e_pad, 16), jnp.float32)],
      mesh=mesh,
      compiler_params=pltpu.CompilerParams(use_tc_tiling_on_sc=False),
      scratch_types=[
          pltpu.VMEM_SHARED((n_pad, 16), jnp.float32),
          pltpu.VMEM((nchunk, cb), jnp.int32),
          pltpu.VMEM((nchunk, cb), jnp.int32),
          pltpu.VMEM((nchunk, cb), jnp.int32),
          pltpu.VMEM((cb, 16), jnp.float32),
          pltpu.VMEM((cb, 16), jnp.float32),
          pltpu.VMEM((cb, 16), jnp.float32),
          pltpu.VMEM((cb, 16), jnp.float32),
          pltpu.VMEM((cb, 16), jnp.float32),
          pltpu.VMEM((cb, 16), jnp.float32),
          pltpu.SemaphoreType.DMA,
          pltpu.SemaphoreType.DMA,
          pltpu.SemaphoreType.DMA,
          pltpu.SemaphoreType.DMA,
      ],
  )
  def k(pcs_hbm, rc2_hbm, rd_hbm, srcA_hbm, dst2_hbm, eidx_hbm, cc_out,
        tab, sidx, didx, eidx, g0, g1, u0, u1, rcb, rdb,
        s0, s1, w0, w1):
    c = lax.axis_index("c")
    s = lax.axis_index("s")

    _zero_rows(g0, cb, 16)
    for kk in range(nblk):
      pltpu.sync_copy(g0, tab.at[pl.ds(s * rpt + kk * cb, cb)])

    pltpu.sync_copy(srcA_hbm.at[c * _NS + s], sidx)
    pltpu.sync_copy(dst2_hbm.at[s], didx)
    pltpu.sync_copy(eidx_hbm.at[c * _NS + s], eidx)
    plsc.subcore_barrier()

    bufs = (g0, g1)
    sems = (s0, s1)

    def start(j, b):
      pltpu.async_copy(pcs_hbm.at[sidx.at[j]], bufs[b], sems[b])

    def finish(j, b):
      pltpu.make_async_copy(pcs_hbm.at[sidx.at[j]], bufs[b], sems[b]).wait()
      pltpu.sync_copy(bufs[b], tab.at[didx.at[j]], add=True)

    start(0, 0)

    @pl.loop(0, nchunk - 2, step=2)
    def seg_body(j):
      start(j + 1, 1)
      finish(j, 0)
      start(j + 2, 0)
      finish(j + 1, 1)

    start(nchunk - 1, 1)
    finish(nchunk - 2, 0)
    finish(nchunk - 1, 1)
    plsc.subcore_barrier()

    for blk in range(nblk):
      row0 = s * rpt + blk * cb
      pltpu.sync_copy(tab.at[pl.ds(row0, cb)], g0)
      pltpu.sync_copy(rc2_hbm.at[pl.ds(c * n_pad + row0, cb)], rcb)
      pltpu.sync_copy(rd_hbm.at[pl.ds(row0, cb)], rdb)

      def rowf(i, cacc):
        u0[i, pl.ds(0, 16)] = (g0[i, pl.ds(0, 16)] * rdb[i, pl.ds(0, 16)]
                               + rcb[i, pl.ds(0, 16)])
        return cacc

      lax.fori_loop(0, cb, rowf, 0)
      pltpu.sync_copy(u0, tab.at[pl.ds(row0, cb)])
    plsc.subcore_barrier()

    ubufs = (u0, u1)
    gsems = (s0, s1)
    wsems = (w0, w1)

    def start_gather(j, p):
      pltpu.async_copy(tab.at[eidx.at[j]], ubufs[p], gsems[p])

    def wait_gather(j, p):
      pltpu.make_async_copy(tab.at[eidx.at[j]], ubufs[p], gsems[p]).wait()

    def obase(j):
      return c * e_pad + s * epw + j * cb

    def start_write(j, p):
      pltpu.async_copy(ubufs[p], cc_out.at[pl.ds(obase(j), cb)], wsems[p])

    def wait_write(j, p):
      pltpu.make_async_copy(ubufs[p], cc_out.at[pl.ds(obase(j), cb)],
                            wsems[p]).wait()

    start_gather(0, 0)
    start_gather(1, 1)
    wait_gather(0, 0)
    start_write(0, 0)

    @pl.loop(1, nchunk - 1, step=2)
    def body(j):
      wait_write(j - 1, 0)
      start_gather(j + 1, 0)
      wait_gather(j, 1)
      start_write(j, 1)
      wait_write(j, 1)
      start_gather(j + 2, 1)
      wait_gather(j + 1, 0)
      start_write(j + 1, 0)

    wait_write(nchunk - 2, 0)
    wait_gather(nchunk - 1, 1)
    start_write(nchunk - 1, 1)
    wait_write(nchunk - 1, 1)

  return k(pcs, rc2, rd, srcA, dst2, eidx2)[0]


def kernel(x, edge_index, W1l, b1l, W1r, W2l, b2l, W2r, W3, b3, W4, b4):
  n, d_in = x.shape
  e = edge_index.shape[1]
  d_hid = W1l.shape[1]

  e_pad = _cdiv(e, 16384) * 16384
  epw = e_pad // _NW
  n_pad = _cdiv(n + 1, _NS * _CB) * _NS * _CB

  src = edge_index[0]
  dst = edge_index[1]
  pad = e_pad - e
  src_p = jnp.concatenate([src, jnp.zeros((pad,), jnp.int32)])
  dst_p = jnp.concatenate([dst, jnp.full((pad,), n, jnp.int32)])
  src3 = src_p.reshape(_NW, epw // _CB, _CB)
  dst3 = dst_p.reshape(_NW, epw // _CB, _CB)

  eps = e_pad // _NS
  src2 = src_p.reshape(_NS, eps // _CB, _CB)
  dst2 = dst_p.reshape(_NS, eps // _CB, _CB)
  srcA = jnp.concatenate([src2, src2 + n_pad], axis=0)
  eidx2 = jnp.concatenate([src2, dst2], axis=0)

  p1, r1b = _mm2(x, W1l, W1r, b1l.reshape(1, d_hid), rb=2000)
  agg1, deg = _seg_sum(p1, src3, dst3, n_pad, with_deg=True)

  pcs3, rc3, rd = _combine_proj(agg1, deg, r1b, W2l, W2r, W3,
                                b2l.reshape(1, 16), b3.reshape(1, 16),
                                n_pad, rb=2048)

  ccat = _fused_uv(pcs3.reshape(2 * n_pad, 16), rc3.reshape(2 * n_pad, 16),
                   rd, srcA, dst2, eidx2, n, n_pad)
  sel = (jnp.arange(128, dtype=jnp.int32)[:, None] // 16
         == jnp.arange(8, dtype=jnp.int32)[None, :]).astype(jnp.float32)
  out2 = _decode(ccat.reshape(-1, 128),
                 jnp.tile(W4.reshape(-1), 8).reshape(1, 128),
                 sel, b4.reshape(1, 1), rb8=4096)
  return out2.reshape(-1)[:e]
